# baseline (speedup 1.0000x reference)
import functools

import jax
import jax.numpy as jnp
from jax import lax
from jax.experimental import pallas as pl
from jax.experimental.pallas import tpu as pltpu
from jax.experimental.pallas import tpu_sc as plsc

_TOPK = 32
_NC = 8


def _proj_body(q_ref, wq_ref, qm_ref):
    q = lax.dot_general(q_ref[0], wq_ref[...], (((1,), (1,)), ((), ())),
                        preferred_element_type=jnp.float32,
                        precision=lax.Precision.DEFAULT)
    s_len = q.shape[0]
    qm_ref[0] = jnp.sum(q, axis=0, keepdims=True) * (1.0 / s_len)


def _scores_topk_body(qm_ref, wk_ref, keys_ref, idx_ref, w_ref, cv_scr,
                      ci_scr, *, m_total, scale):
    b = pl.program_id(0)
    j = pl.program_id(1)
    nj = pl.num_programs(1)
    k = lax.dot_general(keys_ref[0], wk_ref[...], (((1,), (1,)), ((), ())),
                        preferred_element_type=jnp.float32,
                        precision=lax.Precision.DEFAULT)
    s = lax.dot_general(qm_ref[0], k, (((1,), (1,)), ((), ())),
                        preferred_element_type=jnp.float32,
                        precision=lax.Precision.DEFAULT)[0] * scale
    ch = s.shape[0]
    rows = ch // 128
    sc0 = s.reshape(rows, 128)
    flat = (j * ch
            + lax.broadcasted_iota(jnp.int32, (rows, 128), 0) * 128
            + lax.broadcasted_iota(jnp.int32, (rows, 128), 1))
    lane = lax.broadcasted_iota(jnp.int32, (1, 128), 1)
    neg = jnp.float32(-jnp.inf)
    big = jnp.int32(2 ** 30)

    def cbody(i, carry):
        sc, tv, ti = carry
        m = jnp.max(sc)
        idx = jnp.min(jnp.where(sc == m, flat, big))
        tv = jnp.where(lane == i, m, tv)
        ti = jnp.where(lane == i, idx, ti)
        sc = jnp.where(flat == idx, neg, sc)
        return sc, tv, ti

    tv0 = jnp.full((1, 128), neg, jnp.float32)
    ti0 = jnp.zeros((1, 128), jnp.int32)
    _, tv, ti = lax.fori_loop(0, _TOPK, cbody, (sc0, tv0, ti0))
    cv_scr[pl.ds(j, 1)] = tv[None]
    ci_scr[pl.ds(j, 1)] = ti[None]

    @pl.when(j == nj - 1)
    def _():
        cv = cv_scr[...].reshape(nj, 128)
        ci = ci_scr[...].reshape(nj, 128)

        def gbody(i, carry):
            sc, tv2, ti2 = carry
            m = jnp.max(sc)
            idx = jnp.min(jnp.where(sc == m, ci, big))
            tv2 = jnp.where(lane == i, m, tv2)
            ti2 = jnp.where(lane == i, idx, ti2)
            sc = jnp.where((sc == m) & (ci == idx), neg, sc)
            return sc, tv2, ti2

        _, tv2, ti2 = lax.fori_loop(0, _TOPK, gbody, (cv, tv0, ti0))

        valid = lane < _TOPK
        mx = jnp.max(jnp.where(valid, tv2, neg))
        e = jnp.where(valid, jnp.exp(tv2 - mx), jnp.float32(0.0))
        w = e / jnp.sum(e)
        idx_ref[0] = ti2 + b * m_total
        w_ref[0] = w


@functools.lru_cache(maxsize=None)
def _make_sc_gather(n_rows, h):
    n_workers = 16
    rpw = n_rows // n_workers
    mesh = plsc.VectorSubcoreMesh(core_axis_name="c", subcore_axis_name="s")

    @functools.partial(
        pl.kernel, mesh=mesh,
        out_type=jax.ShapeDtypeStruct((n_rows, h), jnp.float32),
        scratch_types=[
            pltpu.VMEM((rpw,), jnp.int32),
            pltpu.VMEM((rpw, h), jnp.float32),
            pltpu.VMEM((rpw, 16), jnp.float32),
            pltpu.SemaphoreType.DMA,
        ],
    )
    def gather_k(values_hbm, idx_hbm, wrep_hbm, out_hbm, idx_v, rows_v, w_v, sem):
        wid = lax.axis_index("s") * 2 + lax.axis_index("c")

        @pl.when(wid < n_workers)
        def _():
            base = wid * rpw
            pltpu.sync_copy(idx_hbm.at[pl.ds(base, rpw)], idx_v)
            pltpu.sync_copy(wrep_hbm.at[pl.ds(base, rpw)], w_v)
            pltpu.async_copy(values_hbm.at[idx_v], rows_v, sem).wait()
            for r in range(rpw):
                wv = w_v[r, :]

                def mul_body(c, carry, r=r, wv=wv):
                    off = c * 16
                    rows_v[r, pl.ds(off, 16)] = rows_v[r, pl.ds(off, 16)] * wv
                    return carry

                lax.fori_loop(0, h // 16, mul_body, 0)
            pltpu.sync_copy(rows_v, out_hbm.at[pl.ds(base, rpw)])

    return gather_k


def kernel(query, memory_keys, memory_values, Wq, Wk):
    B, S, H = query.shape
    M = memory_keys.shape[1]
    KD = Wq.shape[0]
    scale = KD ** (-0.5)
    ch = M // _NC

    qm = pl.pallas_call(
        _proj_body,
        grid=(B,),
        in_specs=[
            pl.BlockSpec((1, S, H), lambda b: (b, 0, 0)),
            pl.BlockSpec((KD, H), lambda b: (0, 0)),
        ],
        out_specs=pl.BlockSpec((1, 1, KD), lambda b: (b, 0, 0)),
        out_shape=jax.ShapeDtypeStruct((B, 1, KD), jnp.float32),
    )(query, Wq)

    idx_pad, w_pad = pl.pallas_call(
        functools.partial(_scores_topk_body, m_total=M, scale=scale),
        grid=(B, _NC),
        in_specs=[
            pl.BlockSpec((1, 1, KD), lambda b, j: (b, 0, 0)),
            pl.BlockSpec((KD, H), lambda b, j: (0, 0)),
            pl.BlockSpec((1, ch, H), lambda b, j: (b, j, 0)),
        ],
        out_specs=[
            pl.BlockSpec((1, 1, 128), lambda b, j: (b, 0, 0)),
            pl.BlockSpec((1, 1, 128), lambda b, j: (b, 0, 0)),
        ],
        out_shape=[
            jax.ShapeDtypeStruct((B, 1, 128), jnp.int32),
            jax.ShapeDtypeStruct((B, 1, 128), jnp.float32),
        ],
        scratch_shapes=[pltpu.VMEM((_NC, 1, 128), jnp.float32),
                        pltpu.VMEM((_NC, 1, 128), jnp.int32)],
    )(qm, Wk, memory_keys)

    idx_flat = idx_pad[:, 0, :_TOPK].reshape(B * _TOPK)
    w_flat = w_pad[:, 0, :_TOPK].reshape(B * _TOPK)
    wrep = jnp.broadcast_to(w_flat[:, None], (B * _TOPK, 16))
    values_flat = memory_values.reshape(B * M, H)

    out = _make_sc_gather(B * _TOPK, H)(values_flat, idx_flat, wrep)
    return out.reshape(B, _TOPK, H)


# speedup vs baseline: 2.5051x; 2.5051x over previous
import functools

import jax
import jax.numpy as jnp
from jax import lax
from jax.experimental import pallas as pl
from jax.experimental.pallas import tpu as pltpu
from jax.experimental.pallas import tpu_sc as plsc

_TOPK = 32
_NC = 8


def _proj_body(q_ref, wq_ref, qm_ref):
    q = lax.dot_general(q_ref[0], wq_ref[...], (((1,), (1,)), ((), ())),
                        preferred_element_type=jnp.float32,
                        precision=lax.Precision.DEFAULT)
    s_len = q.shape[0]
    qm_ref[0] = jnp.sum(q, axis=0, keepdims=True) * (1.0 / s_len)


def _scores_body(qm_ref, wk_ref, keys_ref, s_ref, *, scale):
    k = lax.dot_general(keys_ref[0], wk_ref[...], (((1,), (1,)), ((), ())),
                        preferred_element_type=jnp.float32,
                        precision=lax.Precision.DEFAULT)
    s = lax.dot_general(qm_ref[0], k, (((1,), (1,)), ((), ())),
                        preferred_element_type=jnp.float32,
                        precision=lax.Precision.DEFAULT)[0] * scale
    s_ref[0] = s.reshape(s.shape[0] // 128, 128)


def _topk_body(s_ref, idx_ref, w_ref, *, m_total, n_batch):
    rows = s_ref.shape[1]
    lane = lax.broadcasted_iota(jnp.int32, (1, 128), 1)
    flat = (lax.broadcasted_iota(jnp.int32, (rows, 128), 0) * 128
            + lax.broadcasted_iota(jnp.int32, (rows, 128), 1))
    neg = jnp.float32(-jnp.inf)
    big = jnp.int32(2 ** 30)

    def body(i, carry):
        scs, tv, ti = carry
        out = []
        for b in range(n_batch):
            m = jnp.max(scs[b])
            idx = jnp.min(jnp.where(scs[b] == m, flat, big))
            tv = jnp.where(lane == b * _TOPK + i, m, tv)
            ti = jnp.where(lane == b * _TOPK + i, idx + b * m_total, ti)
            out.append(jnp.where(flat == idx, neg, scs[b]))
        return tuple(out), tv, ti

    tv0 = jnp.full((1, 128), neg, jnp.float32)
    ti0 = jnp.zeros((1, 128), jnp.int32)
    scs0 = tuple(s_ref[b] for b in range(n_batch))
    _, tv, ti = lax.fori_loop(0, _TOPK, body, (scs0, tv0, ti0))

    seg = lane // _TOPK
    e = tv
    mxv = jnp.zeros((1, 128), jnp.float32)
    for b in range(n_batch):
        mb = jnp.max(jnp.where(seg == b, tv, neg))
        mxv = jnp.where(seg == b, mb, mxv)
    e = jnp.exp(tv - mxv)
    dnv = jnp.zeros((1, 128), jnp.float32)
    for b in range(n_batch):
        db = jnp.sum(jnp.where(seg == b, e, jnp.float32(0.0)))
        dnv = jnp.where(seg == b, db, dnv)
    idx_ref[...] = ti
    w_ref[...] = e / dnv


@functools.lru_cache(maxsize=None)
def _make_sc_gather(n_rows, h):
    n_workers = 16
    rpw = n_rows // n_workers
    mesh = plsc.VectorSubcoreMesh(core_axis_name="c", subcore_axis_name="s")

    @functools.partial(
        pl.kernel, mesh=mesh,
        out_type=jax.ShapeDtypeStruct((n_rows, h), jnp.float32),
        scratch_types=[
            pltpu.VMEM((rpw,), jnp.int32),
            pltpu.VMEM((rpw, h), jnp.float32),
            pltpu.VMEM((rpw, 16), jnp.float32),
            pltpu.SemaphoreType.DMA,
        ],
    )
    def gather_k(values_hbm, idx_hbm, wrep_hbm, out_hbm, idx_v, rows_v, w_v, sem):
        wid = lax.axis_index("s") * 2 + lax.axis_index("c")

        @pl.when(wid < n_workers)
        def _():
            base = wid * rpw
            pltpu.sync_copy(idx_hbm.at[pl.ds(base, rpw)], idx_v)
            pltpu.sync_copy(wrep_hbm.at[pl.ds(base, rpw)], w_v)
            pltpu.async_copy(values_hbm.at[idx_v], rows_v, sem).wait()
            for r in range(rpw):
                wv = w_v[r, :]

                def mul_body(c, carry, r=r, wv=wv):
                    off = c * 16
                    rows_v[r, pl.ds(off, 16)] = rows_v[r, pl.ds(off, 16)] * wv
                    return carry

                lax.fori_loop(0, h // 16, mul_body, 0)
            pltpu.sync_copy(rows_v, out_hbm.at[pl.ds(base, rpw)])

    return gather_k


def kernel(query, memory_keys, memory_values, Wq, Wk):
    B, S, H = query.shape
    M = memory_keys.shape[1]
    KD = Wq.shape[0]
    scale = KD ** (-0.5)
    ch = M // _NC

    qm = pl.pallas_call(
        _proj_body,
        grid=(B,),
        in_specs=[
            pl.BlockSpec((1, S, H), lambda b: (b, 0, 0)),
            pl.BlockSpec((KD, H), lambda b: (0, 0)),
        ],
        out_specs=pl.BlockSpec((1, 1, KD), lambda b: (b, 0, 0)),
        out_shape=jax.ShapeDtypeStruct((B, 1, KD), jnp.float32),
    )(query, Wq)

    scores = pl.pallas_call(
        functools.partial(_scores_body, scale=scale),
        grid=(B, _NC),
        in_specs=[
            pl.BlockSpec((1, 1, KD), lambda b, j: (b, 0, 0)),
            pl.BlockSpec((KD, H), lambda b, j: (0, 0)),
            pl.BlockSpec((1, ch, H), lambda b, j: (b, j, 0)),
        ],
        out_specs=pl.BlockSpec((1, ch // 128, 128), lambda b, j: (b, j, 0)),
        out_shape=jax.ShapeDtypeStruct((B, M // 128, 128), jnp.float32),
    )(qm, Wk, memory_keys)

    idx_all, w_all = pl.pallas_call(
        functools.partial(_topk_body, m_total=M, n_batch=B),
        out_shape=[
            jax.ShapeDtypeStruct((1, 128), jnp.int32),
            jax.ShapeDtypeStruct((1, 128), jnp.float32),
        ],
    )(scores)

    idx_flat = idx_all.reshape(B * _TOPK)
    wrep = jnp.broadcast_to(w_all.reshape(B * _TOPK, 1), (B * _TOPK, 16))
    values_flat = memory_values.reshape(B * M, H)

    out = _make_sc_gather(B * _TOPK, H)(values_flat, idx_flat, wrep)
    return out.reshape(B, _TOPK, H)


# speedup vs baseline: 2.8238x; 1.1272x over previous
import functools

import jax
import jax.numpy as jnp
from jax import lax
from jax.experimental import pallas as pl
from jax.experimental.pallas import tpu as pltpu
from jax.experimental.pallas import tpu_sc as plsc

_TOPK = 32
_NC = 8


def _proj_body(q_ref, wq_ref, qm_ref):
    q = lax.dot_general(q_ref[0].astype(jnp.bfloat16),
                        wq_ref[...].astype(jnp.bfloat16),
                        (((1,), (1,)), ((), ())),
                        preferred_element_type=jnp.float32)
    s_len = q.shape[0]
    qm_ref[0] = jnp.sum(q, axis=0, keepdims=True) * (1.0 / s_len)


def _scores_body(qm_ref, wk_ref, keys_ref, s_ref, *, scale):
    k = lax.dot_general(keys_ref[0].astype(jnp.bfloat16),
                        wk_ref[...].astype(jnp.bfloat16),
                        (((1,), (1,)), ((), ())),
                        preferred_element_type=jnp.float32)
    s = lax.dot_general(qm_ref[0].astype(jnp.bfloat16),
                        k.astype(jnp.bfloat16),
                        (((1,), (1,)), ((), ())),
                        preferred_element_type=jnp.float32)[0] * scale
    s_ref[0] = s.reshape(s.shape[0] // 128, 128)


def _topk_body(s_ref, idx_ref, w_ref, *, m_total, n_batch):
    rows = s_ref.shape[1]
    lane = lax.broadcasted_iota(jnp.int32, (1, 128), 1)
    flat = (lax.broadcasted_iota(jnp.int32, (rows, 128), 0) * 128
            + lax.broadcasted_iota(jnp.int32, (rows, 128), 1))
    neg = jnp.float32(-jnp.inf)
    big = jnp.int32(2 ** 30)

    def body(i, carry):
        scs, tv, ti = carry
        out = []
        for b in range(n_batch):
            m = jnp.max(scs[b])
            idx = jnp.min(jnp.where(scs[b] == m, flat, big))
            tv = jnp.where(lane == b * _TOPK + i, m, tv)
            ti = jnp.where(lane == b * _TOPK + i, idx + b * m_total, ti)
            out.append(jnp.where(flat == idx, neg, scs[b]))
        return tuple(out), tv, ti

    tv0 = jnp.full((1, 128), neg, jnp.float32)
    ti0 = jnp.zeros((1, 128), jnp.int32)
    scs0 = tuple(s_ref[b] for b in range(n_batch))
    _, tv, ti = lax.fori_loop(0, _TOPK, body, (scs0, tv0, ti0))

    seg = lane // _TOPK
    e = tv
    mxv = jnp.zeros((1, 128), jnp.float32)
    for b in range(n_batch):
        mb = jnp.max(jnp.where(seg == b, tv, neg))
        mxv = jnp.where(seg == b, mb, mxv)
    e = jnp.exp(tv - mxv)
    dnv = jnp.zeros((1, 128), jnp.float32)
    for b in range(n_batch):
        db = jnp.sum(jnp.where(seg == b, e, jnp.float32(0.0)))
        dnv = jnp.where(seg == b, db, dnv)
    idx_ref[...] = ti
    w_ref[...] = e / dnv


@functools.lru_cache(maxsize=None)
def _make_sc_gather(n_rows, h):
    n_workers = 16
    rpw = n_rows // n_workers
    mesh = plsc.VectorSubcoreMesh(core_axis_name="c", subcore_axis_name="s")

    @functools.partial(
        pl.kernel, mesh=mesh,
        out_type=jax.ShapeDtypeStruct((n_rows, h), jnp.float32),
        scratch_types=[
            pltpu.VMEM((rpw,), jnp.int32),
            pltpu.VMEM((rpw, h), jnp.float32),
            pltpu.VMEM((rpw, 16), jnp.float32),
            pltpu.SemaphoreType.DMA,
        ],
    )
    def gather_k(values_hbm, idx_hbm, wrep_hbm, out_hbm, idx_v, rows_v, w_v, sem):
        wid = lax.axis_index("s") * 2 + lax.axis_index("c")

        @pl.when(wid < n_workers)
        def _():
            base = wid * rpw
            pltpu.sync_copy(idx_hbm.at[pl.ds(base, rpw)], idx_v)
            pltpu.sync_copy(wrep_hbm.at[pl.ds(base, rpw)], w_v)
            pltpu.async_copy(values_hbm.at[idx_v], rows_v, sem).wait()
            for r in range(rpw):
                wv = w_v[r, :]

                def mul_body(c, carry, r=r, wv=wv):
                    off = c * 16
                    rows_v[r, pl.ds(off, 16)] = rows_v[r, pl.ds(off, 16)] * wv
                    return carry

                lax.fori_loop(0, h // 16, mul_body, 0)
            pltpu.sync_copy(rows_v, out_hbm.at[pl.ds(base, rpw)])

    return gather_k


def kernel(query, memory_keys, memory_values, Wq, Wk):
    B, S, H = query.shape
    M = memory_keys.shape[1]
    KD = Wq.shape[0]
    scale = KD ** (-0.5)
    ch = M // _NC

    qm = pl.pallas_call(
        _proj_body,
        grid=(B,),
        in_specs=[
            pl.BlockSpec((1, S, H), lambda b: (b, 0, 0)),
            pl.BlockSpec((KD, H), lambda b: (0, 0)),
        ],
        out_specs=pl.BlockSpec((1, 1, KD), lambda b: (b, 0, 0)),
        out_shape=jax.ShapeDtypeStruct((B, 1, KD), jnp.float32),
    )(query, Wq)

    scores = pl.pallas_call(
        functools.partial(_scores_body, scale=scale),
        grid=(B, _NC),
        in_specs=[
            pl.BlockSpec((1, 1, KD), lambda b, j: (b, 0, 0)),
            pl.BlockSpec((KD, H), lambda b, j: (0, 0)),
            pl.BlockSpec((1, ch, H), lambda b, j: (b, j, 0)),
        ],
        out_specs=pl.BlockSpec((1, ch // 128, 128), lambda b, j: (b, j, 0)),
        out_shape=jax.ShapeDtypeStruct((B, M // 128, 128), jnp.float32),
    )(qm, Wk, memory_keys)

    idx_all, w_all = pl.pallas_call(
        functools.partial(_topk_body, m_total=M, n_batch=B),
        out_shape=[
            jax.ShapeDtypeStruct((1, 128), jnp.int32),
            jax.ShapeDtypeStruct((1, 128), jnp.float32),
        ],
    )(scores)

    idx_flat = idx_all.reshape(B * _TOPK)
    wrep = jnp.broadcast_to(w_all.reshape(B * _TOPK, 1), (B * _TOPK, 16))
    values_flat = memory_values.reshape(B * M, H)

    out = _make_sc_gather(B * _TOPK, H)(values_flat, idx_flat, wrep)
    return out.reshape(B, _TOPK, H)


# speedup vs baseline: 2.9464x; 1.0434x over previous
import functools

import jax
import jax.numpy as jnp
from jax import lax
from jax.experimental import pallas as pl
from jax.experimental.pallas import tpu as pltpu
from jax.experimental.pallas import tpu_sc as plsc

_TOPK = 32
_NC = 4


def _proj_body(q_ref, wq_ref, qm_ref):
    q = lax.dot_general(q_ref[0].astype(jnp.bfloat16),
                        wq_ref[...].astype(jnp.bfloat16),
                        (((1,), (1,)), ((), ())),
                        preferred_element_type=jnp.float32)
    s_len = q.shape[0]
    qm_ref[0] = jnp.sum(q, axis=0, keepdims=True) * (1.0 / s_len)


def _scores_body(qm_ref, wk_ref, keys_ref, s_ref, *, scale):
    k = lax.dot_general(keys_ref[0].astype(jnp.bfloat16),
                        wk_ref[...].astype(jnp.bfloat16),
                        (((1,), (1,)), ((), ())),
                        preferred_element_type=jnp.float32)
    s = lax.dot_general(qm_ref[0].astype(jnp.bfloat16),
                        k.astype(jnp.bfloat16),
                        (((1,), (1,)), ((), ())),
                        preferred_element_type=jnp.float32)[0] * scale
    s_ref[0] = s.reshape(s.shape[0] // 128, 128)


def _topk_body(s_ref, idx_ref, w_ref, *, m_total, n_batch):
    rows = s_ref.shape[1]
    lane = lax.broadcasted_iota(jnp.int32, (1, 128), 1)
    flat = (lax.broadcasted_iota(jnp.int32, (rows, 128), 0) * 128
            + lax.broadcasted_iota(jnp.int32, (rows, 128), 1))
    neg = jnp.float32(-jnp.inf)
    big = jnp.int32(2 ** 30)

    def body(i, carry):
        scs, tv, ti = carry
        out = []
        for b in range(n_batch):
            m = jnp.max(scs[b])
            idx = jnp.min(jnp.where(scs[b] == m, flat, big))
            tv = jnp.where(lane == b * _TOPK + i, m, tv)
            ti = jnp.where(lane == b * _TOPK + i, idx + b * m_total, ti)
            out.append(jnp.where(flat == idx, neg, scs[b]))
        return tuple(out), tv, ti

    tv0 = jnp.full((1, 128), neg, jnp.float32)
    ti0 = jnp.zeros((1, 128), jnp.int32)
    scs0 = tuple(s_ref[b] for b in range(n_batch))
    _, tv, ti = lax.fori_loop(0, _TOPK, body, (scs0, tv0, ti0))

    seg = lane // _TOPK
    e = tv
    mxv = jnp.zeros((1, 128), jnp.float32)
    for b in range(n_batch):
        mb = jnp.max(jnp.where(seg == b, tv, neg))
        mxv = jnp.where(seg == b, mb, mxv)
    e = jnp.exp(tv - mxv)
    dnv = jnp.zeros((1, 128), jnp.float32)
    for b in range(n_batch):
        db = jnp.sum(jnp.where(seg == b, e, jnp.float32(0.0)))
        dnv = jnp.where(seg == b, db, dnv)
    idx_ref[...] = ti
    w_ref[...] = e / dnv


@functools.lru_cache(maxsize=None)
def _make_sc_gather(n_rows, h):
    n_workers = 16
    rpw = n_rows // n_workers
    mesh = plsc.VectorSubcoreMesh(core_axis_name="c", subcore_axis_name="s")

    @functools.partial(
        pl.kernel, mesh=mesh,
        out_type=jax.ShapeDtypeStruct((n_rows, h), jnp.float32),
        scratch_types=[
            pltpu.VMEM((rpw,), jnp.int32),
            pltpu.VMEM((rpw, h), jnp.float32),
            pltpu.VMEM((rpw, 16), jnp.float32),
            pltpu.SemaphoreType.DMA,
        ],
    )
    def gather_k(values_hbm, idx_hbm, wrep_hbm, out_hbm, idx_v, rows_v, w_v, sem):
        wid = lax.axis_index("s") * 2 + lax.axis_index("c")

        @pl.when(wid < n_workers)
        def _():
            base = wid * rpw
            pltpu.sync_copy(idx_hbm.at[pl.ds(base, rpw)], idx_v)
            pltpu.sync_copy(wrep_hbm.at[pl.ds(base, rpw)], w_v)
            pltpu.async_copy(values_hbm.at[idx_v], rows_v, sem).wait()
            for r in range(rpw):
                wv = w_v[r, :]

                def mul_body(c, carry, r=r, wv=wv):
                    off = c * 16
                    rows_v[r, pl.ds(off, 16)] = rows_v[r, pl.ds(off, 16)] * wv
                    return carry

                lax.fori_loop(0, h // 16, mul_body, 0)
            pltpu.sync_copy(rows_v, out_hbm.at[pl.ds(base, rpw)])

    return gather_k


def kernel(query, memory_keys, memory_values, Wq, Wk):
    B, S, H = query.shape
    M = memory_keys.shape[1]
    KD = Wq.shape[0]
    scale = KD ** (-0.5)
    ch = M // _NC

    qm = pl.pallas_call(
        _proj_body,
        grid=(B,),
        in_specs=[
            pl.BlockSpec((1, S, H), lambda b: (b, 0, 0)),
            pl.BlockSpec((KD, H), lambda b: (0, 0)),
        ],
        out_specs=pl.BlockSpec((1, 1, KD), lambda b: (b, 0, 0)),
        out_shape=jax.ShapeDtypeStruct((B, 1, KD), jnp.float32),
    )(query, Wq)

    scores = pl.pallas_call(
        functools.partial(_scores_body, scale=scale),
        grid=(B, _NC),
        in_specs=[
            pl.BlockSpec((1, 1, KD), lambda b, j: (b, 0, 0)),
            pl.BlockSpec((KD, H), lambda b, j: (0, 0)),
            pl.BlockSpec((1, ch, H), lambda b, j: (b, j, 0)),
        ],
        out_specs=pl.BlockSpec((1, ch // 128, 128), lambda b, j: (b, j, 0)),
        out_shape=jax.ShapeDtypeStruct((B, M // 128, 128), jnp.float32),
    )(qm, Wk, memory_keys)

    idx_all, w_all = pl.pallas_call(
        functools.partial(_topk_body, m_total=M, n_batch=B),
        out_shape=[
            jax.ShapeDtypeStruct((1, 128), jnp.int32),
            jax.ShapeDtypeStruct((1, 128), jnp.float32),
        ],
    )(scores)

    idx_flat = idx_all.reshape(B * _TOPK)
    wrep = jnp.broadcast_to(w_all.reshape(B * _TOPK, 1), (B * _TOPK, 16))
    values_flat = memory_values.reshape(B * M, H)

    out = _make_sc_gather(B * _TOPK, H)(values_flat, idx_flat, wrep)
    return out.reshape(B, _TOPK, H)


# speedup vs baseline: 3.0211x; 1.0254x over previous
import functools

import jax
import jax.numpy as jnp
from jax import lax
from jax.experimental import pallas as pl
from jax.experimental.pallas import tpu as pltpu
from jax.experimental.pallas import tpu_sc as plsc

_TOPK = 32
_NC = 4


def _proj_body(q_ref, wq_ref, qm_ref):
    q = lax.dot_general(q_ref[0].astype(jnp.bfloat16),
                        wq_ref[...].astype(jnp.bfloat16),
                        (((1,), (1,)), ((), ())),
                        preferred_element_type=jnp.float32)
    s_len = q.shape[0]
    qm_ref[0] = jnp.sum(q, axis=0, keepdims=True) * (1.0 / s_len)


def _scores_body(qm_ref, wk_ref, keys_ref, s_ref, *, scale):
    k = lax.dot_general(keys_ref[0].astype(jnp.bfloat16),
                        wk_ref[...].astype(jnp.bfloat16),
                        (((1,), (1,)), ((), ())),
                        preferred_element_type=jnp.float32)
    s = lax.dot_general(qm_ref[0].astype(jnp.bfloat16),
                        k.astype(jnp.bfloat16),
                        (((1,), (1,)), ((), ())),
                        preferred_element_type=jnp.float32)[0] * scale
    s_ref[0] = s.reshape(s.shape[0] // 128, 128)


def _topk_body(s_ref, idx_ref, w_ref, *, m_total, n_batch):
    rows = s_ref.shape[1]
    lane = lax.broadcasted_iota(jnp.int32, (1, 128), 1)
    flat = (lax.broadcasted_iota(jnp.int32, (rows, 128), 0) * 128
            + lax.broadcasted_iota(jnp.int32, (rows, 128), 1))
    neg = jnp.float32(-jnp.inf)
    big = jnp.int32(2 ** 30)

    def body(i, carry):
        scs, tv, ti = carry
        out = []
        for b in range(n_batch):
            m = jnp.max(scs[b])
            idx = jnp.min(jnp.where(scs[b] == m, flat, big))
            tv = jnp.where(lane == b * _TOPK + i, m, tv)
            ti = jnp.where(lane == b * _TOPK + i, idx + b * m_total, ti)
            out.append(jnp.where(flat == idx, neg, scs[b]))
        return tuple(out), tv, ti

    tv0 = jnp.full((1, 128), neg, jnp.float32)
    ti0 = jnp.zeros((1, 128), jnp.int32)
    scs0 = tuple(s_ref[b] for b in range(n_batch))
    _, tv, ti = lax.fori_loop(0, _TOPK, body, (scs0, tv0, ti0))

    seg = lane // _TOPK
    e = tv
    mxv = jnp.zeros((1, 128), jnp.float32)
    for b in range(n_batch):
        mb = jnp.max(jnp.where(seg == b, tv, neg))
        mxv = jnp.where(seg == b, mb, mxv)
    e = jnp.exp(tv - mxv)
    dnv = jnp.zeros((1, 128), jnp.float32)
    for b in range(n_batch):
        db = jnp.sum(jnp.where(seg == b, e, jnp.float32(0.0)))
        dnv = jnp.where(seg == b, db, dnv)
    idx_ref[...] = ti
    w_ref[...] = e / dnv


@functools.lru_cache(maxsize=None)
def _make_sc_gather(n_rows, h):
    n_workers = 16
    rpw = n_rows // n_workers
    mesh = plsc.VectorSubcoreMesh(core_axis_name="c", subcore_axis_name="s")

    @functools.partial(
        pl.kernel, mesh=mesh,
        out_type=jax.ShapeDtypeStruct((n_rows, h), jnp.float32),
        scratch_types=[
            pltpu.VMEM((rpw,), jnp.int32),
            pltpu.VMEM((rpw, h), jnp.float32),
            pltpu.VMEM((rpw, 16), jnp.float32),
            pltpu.SemaphoreType.DMA,
        ],
    )
    def gather_k(values_hbm, idx_hbm, wrep_hbm, out_hbm, idx_v, rows_v, w_v, sem):
        wid = lax.axis_index("s") * 2 + lax.axis_index("c")

        @pl.when(wid < n_workers)
        def _():
            base = wid * rpw
            pltpu.sync_copy(idx_hbm.at[pl.ds(base, rpw)], idx_v)
            pltpu.sync_copy(wrep_hbm.at[pl.ds(base, rpw)], w_v)
            pltpu.async_copy(values_hbm.at[idx_v], rows_v, sem).wait()
            for r in range(rpw):
                wv = w_v[r, :]

                def mul_body(c, carry, r=r, wv=wv):
                    base_off = c * 128
                    for u in range(8):
                        off = base_off + u * 16
                        rows_v[r, pl.ds(off, 16)] = rows_v[r, pl.ds(off, 16)] * wv
                    return carry

                lax.fori_loop(0, h // 128, mul_body, 0)
            pltpu.sync_copy(rows_v, out_hbm.at[pl.ds(base, rpw)])

    return gather_k


def kernel(query, memory_keys, memory_values, Wq, Wk):
    B, S, H = query.shape
    M = memory_keys.shape[1]
    KD = Wq.shape[0]
    scale = KD ** (-0.5)
    ch = M // _NC

    qm = pl.pallas_call(
        _proj_body,
        grid=(B,),
        in_specs=[
            pl.BlockSpec((1, S, H), lambda b: (b, 0, 0)),
            pl.BlockSpec((KD, H), lambda b: (0, 0)),
        ],
        out_specs=pl.BlockSpec((1, 1, KD), lambda b: (b, 0, 0)),
        out_shape=jax.ShapeDtypeStruct((B, 1, KD), jnp.float32),
    )(query, Wq)

    scores = pl.pallas_call(
        functools.partial(_scores_body, scale=scale),
        grid=(B, _NC),
        in_specs=[
            pl.BlockSpec((1, 1, KD), lambda b, j: (b, 0, 0)),
            pl.BlockSpec((KD, H), lambda b, j: (0, 0)),
            pl.BlockSpec((1, ch, H), lambda b, j: (b, j, 0)),
        ],
        out_specs=pl.BlockSpec((1, ch // 128, 128), lambda b, j: (b, j, 0)),
        out_shape=jax.ShapeDtypeStruct((B, M // 128, 128), jnp.float32),
    )(qm, Wk, memory_keys)

    idx_all, w_all = pl.pallas_call(
        functools.partial(_topk_body, m_total=M, n_batch=B),
        out_shape=[
            jax.ShapeDtypeStruct((1, 128), jnp.int32),
            jax.ShapeDtypeStruct((1, 128), jnp.float32),
        ],
    )(scores)

    idx_flat = idx_all.reshape(B * _TOPK)
    wrep = jnp.broadcast_to(w_all.reshape(B * _TOPK, 1), (B * _TOPK, 16))
    values_flat = memory_values.reshape(B * M, H)

    out = _make_sc_gather(B * _TOPK, H)(values_flat, idx_flat, wrep)
    return out.reshape(B, _TOPK, H)


# speedup vs baseline: 3.0219x; 1.0003x over previous
import functools

import jax
import jax.numpy as jnp
from jax import lax
from jax.experimental import pallas as pl
from jax.experimental.pallas import tpu as pltpu
from jax.experimental.pallas import tpu_sc as plsc

_TOPK = 32
_NC = 4


def _proj_body(q_ref, wq_ref, qm_ref):
    q = lax.dot_general(q_ref[0].astype(jnp.bfloat16),
                        wq_ref[...].astype(jnp.bfloat16),
                        (((1,), (1,)), ((), ())),
                        preferred_element_type=jnp.float32)
    s_len = q.shape[0]
    qm_ref[0] = jnp.sum(q, axis=0, keepdims=True) * (1.0 / s_len)


def _scores_body(qm_ref, wk_ref, keys_ref, s_ref, *, scale):
    k = lax.dot_general(keys_ref[0].astype(jnp.bfloat16),
                        wk_ref[...].astype(jnp.bfloat16),
                        (((1,), (1,)), ((), ())),
                        preferred_element_type=jnp.float32)
    s = lax.dot_general(qm_ref[0].astype(jnp.bfloat16),
                        k.astype(jnp.bfloat16),
                        (((1,), (1,)), ((), ())),
                        preferred_element_type=jnp.float32)[0] * scale
    s_ref[0] = s.reshape(s.shape[0] // 128, 128)


def _topk_body(s_ref, idx_ref, w_ref, *, m_total, n_batch):
    rows = s_ref.shape[1]
    lane = lax.broadcasted_iota(jnp.int32, (1, 128), 1)
    flat = (lax.broadcasted_iota(jnp.int32, (rows, 128), 0) * 128
            + lax.broadcasted_iota(jnp.int32, (rows, 128), 1))
    neg = jnp.float32(-jnp.inf)
    big = jnp.int32(2 ** 30)

    def body(i, carry):
        scs, tvs, tis = carry
        out, ntv, nti = [], [], []
        for b in range(n_batch):
            m = jnp.max(scs[b])
            idx = jnp.min(jnp.where(scs[b] == m, flat, big))
            ntv.append(jnp.where(lane == b * _TOPK + i, m, tvs[b]))
            nti.append(jnp.where(lane == b * _TOPK + i, idx + b * m_total,
                                 tis[b]))
            out.append(jnp.where(flat == idx, neg, scs[b]))
        return tuple(out), tuple(ntv), tuple(nti)

    tv0 = jnp.full((1, 128), neg, jnp.float32)
    ti0 = jnp.zeros((1, 128), jnp.int32)
    scs0 = tuple(s_ref[b] for b in range(n_batch))
    _, tvs, tis = lax.fori_loop(
        0, _TOPK, body,
        (scs0, (tv0,) * n_batch, (ti0,) * n_batch))
    tv = functools.reduce(jnp.maximum, tvs)
    ti = functools.reduce(jnp.bitwise_or, tis)

    seg = lane // _TOPK
    e = tv
    mxv = jnp.zeros((1, 128), jnp.float32)
    for b in range(n_batch):
        mb = jnp.max(jnp.where(seg == b, tv, neg))
        mxv = jnp.where(seg == b, mb, mxv)
    e = jnp.exp(tv - mxv)
    dnv = jnp.zeros((1, 128), jnp.float32)
    for b in range(n_batch):
        db = jnp.sum(jnp.where(seg == b, e, jnp.float32(0.0)))
        dnv = jnp.where(seg == b, db, dnv)
    idx_ref[...] = ti
    w_ref[...] = e / dnv


@functools.lru_cache(maxsize=None)
def _make_sc_gather(n_rows, h):
    n_workers = 16
    rpw = n_rows // n_workers
    mesh = plsc.VectorSubcoreMesh(core_axis_name="c", subcore_axis_name="s")

    @functools.partial(
        pl.kernel, mesh=mesh,
        out_type=jax.ShapeDtypeStruct((n_rows, h), jnp.float32),
        scratch_types=[
            pltpu.VMEM((rpw,), jnp.int32),
            pltpu.VMEM((rpw, h), jnp.float32),
            pltpu.VMEM((rpw, 16), jnp.float32),
            pltpu.SemaphoreType.DMA,
        ],
    )
    def gather_k(values_hbm, idx_hbm, wrep_hbm, out_hbm, idx_v, rows_v, w_v, sem):
        wid = lax.axis_index("s") * 2 + lax.axis_index("c")

        @pl.when(wid < n_workers)
        def _():
            base = wid * rpw
            pltpu.sync_copy(idx_hbm.at[pl.ds(base, rpw)], idx_v)
            pltpu.sync_copy(wrep_hbm.at[pl.ds(base, rpw)], w_v)
            pltpu.async_copy(values_hbm.at[idx_v], rows_v, sem).wait()
            for r in range(rpw):
                wv = w_v[r, :]

                def mul_body(c, carry, r=r, wv=wv):
                    base_off = c * 128
                    for u in range(8):
                        off = base_off + u * 16
                        rows_v[r, pl.ds(off, 16)] = rows_v[r, pl.ds(off, 16)] * wv
                    return carry

                lax.fori_loop(0, h // 128, mul_body, 0)
            pltpu.sync_copy(rows_v, out_hbm.at[pl.ds(base, rpw)])

    return gather_k


def kernel(query, memory_keys, memory_values, Wq, Wk):
    B, S, H = query.shape
    M = memory_keys.shape[1]
    KD = Wq.shape[0]
    scale = KD ** (-0.5)
    ch = M // _NC

    qm = pl.pallas_call(
        _proj_body,
        grid=(B,),
        in_specs=[
            pl.BlockSpec((1, S, H), lambda b: (b, 0, 0)),
            pl.BlockSpec((KD, H), lambda b: (0, 0)),
        ],
        out_specs=pl.BlockSpec((1, 1, KD), lambda b: (b, 0, 0)),
        out_shape=jax.ShapeDtypeStruct((B, 1, KD), jnp.float32),
    )(query, Wq)

    scores = pl.pallas_call(
        functools.partial(_scores_body, scale=scale),
        grid=(B, _NC),
        in_specs=[
            pl.BlockSpec((1, 1, KD), lambda b, j: (b, 0, 0)),
            pl.BlockSpec((KD, H), lambda b, j: (0, 0)),
            pl.BlockSpec((1, ch, H), lambda b, j: (b, j, 0)),
        ],
        out_specs=pl.BlockSpec((1, ch // 128, 128), lambda b, j: (b, j, 0)),
        out_shape=jax.ShapeDtypeStruct((B, M // 128, 128), jnp.float32),
    )(qm, Wk, memory_keys)

    idx_all, w_all = pl.pallas_call(
        functools.partial(_topk_body, m_total=M, n_batch=B),
        out_shape=[
            jax.ShapeDtypeStruct((1, 128), jnp.int32),
            jax.ShapeDtypeStruct((1, 128), jnp.float32),
        ],
    )(scores)

    idx_flat = idx_all.reshape(B * _TOPK)
    wrep = jnp.broadcast_to(w_all.reshape(B * _TOPK, 1), (B * _TOPK, 16))
    values_flat = memory_values.reshape(B * M, H)

    out = _make_sc_gather(B * _TOPK, H)(values_flat, idx_flat, wrep)
    return out.reshape(B, _TOPK, H)


# speedup vs baseline: 3.0372x; 1.0050x over previous
import functools

import jax
import jax.numpy as jnp
from jax import lax
from jax.experimental import pallas as pl
from jax.experimental.pallas import tpu as pltpu
from jax.experimental.pallas import tpu_sc as plsc

_TOPK = 32
_NC = 4


def _proj_body(q_ref, wq_ref, qm_ref):
    q = lax.dot_general(q_ref[0].astype(jnp.bfloat16),
                        wq_ref[...].astype(jnp.bfloat16),
                        (((1,), (1,)), ((), ())),
                        preferred_element_type=jnp.float32)
    s_len = q.shape[0]
    qm_ref[0] = jnp.sum(q, axis=0, keepdims=True) * (1.0 / s_len)


def _scores_body(qm_ref, wk_ref, keys_ref, s_ref, *, scale):
    k = lax.dot_general(keys_ref[0].astype(jnp.bfloat16),
                        wk_ref[...].astype(jnp.bfloat16),
                        (((1,), (1,)), ((), ())),
                        preferred_element_type=jnp.float32)
    s = lax.dot_general(qm_ref[0].astype(jnp.bfloat16),
                        k.astype(jnp.bfloat16),
                        (((1,), (1,)), ((), ())),
                        preferred_element_type=jnp.float32)[0] * scale
    s_ref[0] = s.reshape(s.shape[0] // 128, 128)


def _topk_body(s_ref, idx_ref, w_ref, wrep_ref, *, m_total, n_batch):
    rows = s_ref.shape[1]
    lane = lax.broadcasted_iota(jnp.int32, (1, 128), 1)
    flat = (lax.broadcasted_iota(jnp.int32, (rows, 128), 0) * 128
            + lax.broadcasted_iota(jnp.int32, (rows, 128), 1))
    neg = jnp.float32(-jnp.inf)
    big = jnp.int32(2 ** 30)

    def body(i, carry):
        scs, tvs, tis = carry
        out, ntv, nti = [], [], []
        for b in range(n_batch):
            m = jnp.max(scs[b])
            idx = jnp.min(jnp.where(scs[b] == m, flat, big))
            ntv.append(jnp.where(lane == b * _TOPK + i, m, tvs[b]))
            nti.append(jnp.where(lane == b * _TOPK + i, idx + b * m_total,
                                 tis[b]))
            out.append(jnp.where(flat == idx, neg, scs[b]))
        return tuple(out), tuple(ntv), tuple(nti)

    tv0 = jnp.full((1, 128), neg, jnp.float32)
    ti0 = jnp.zeros((1, 128), jnp.int32)
    scs0 = tuple(s_ref[b] for b in range(n_batch))
    _, tvs, tis = lax.fori_loop(
        0, _TOPK, body,
        (scs0, (tv0,) * n_batch, (ti0,) * n_batch))
    tv = functools.reduce(jnp.maximum, tvs)
    ti = functools.reduce(jnp.bitwise_or, tis)

    seg = lane // _TOPK
    e = tv
    mxv = jnp.zeros((1, 128), jnp.float32)
    for b in range(n_batch):
        mb = jnp.max(jnp.where(seg == b, tv, neg))
        mxv = jnp.where(seg == b, mb, mxv)
    e = jnp.exp(tv - mxv)
    dnv = jnp.zeros((1, 128), jnp.float32)
    for b in range(n_batch):
        db = jnp.sum(jnp.where(seg == b, e, jnp.float32(0.0)))
        dnv = jnp.where(seg == b, db, dnv)
    idx_ref[...] = ti
    w = e / dnv
    w_ref[...] = w
    wrep_ref[...] = jnp.broadcast_to(jnp.transpose(w, (1, 0)),
                                     wrep_ref.shape)


@functools.lru_cache(maxsize=None)
def _make_sc_gather(n_rows, h):
    n_workers = 16
    rpw = n_rows // n_workers
    mesh = plsc.VectorSubcoreMesh(core_axis_name="c", subcore_axis_name="s")

    @functools.partial(
        pl.kernel, mesh=mesh,
        out_type=jax.ShapeDtypeStruct((n_rows, h), jnp.float32),
        scratch_types=[
            pltpu.VMEM((rpw,), jnp.int32),
            pltpu.VMEM((rpw, h), jnp.float32),
            pltpu.VMEM((rpw, 16), jnp.float32),
            pltpu.SemaphoreType.DMA,
        ],
    )
    def gather_k(values_hbm, idx_hbm, wrep_hbm, out_hbm, idx_v, rows_v, w_v, sem):
        wid = lax.axis_index("s") * 2 + lax.axis_index("c")

        @pl.when(wid < n_workers)
        def _():
            base = wid * rpw
            pltpu.sync_copy(idx_hbm.at[pl.ds(base, rpw)], idx_v)
            pltpu.sync_copy(wrep_hbm.at[pl.ds(base, rpw)], w_v)
            pltpu.async_copy(values_hbm.at[idx_v], rows_v, sem).wait()
            for r in range(rpw):
                wv = w_v[r, :]

                def mul_body(c, carry, r=r, wv=wv):
                    base_off = c * 128
                    for u in range(8):
                        off = base_off + u * 16
                        rows_v[r, pl.ds(off, 16)] = rows_v[r, pl.ds(off, 16)] * wv
                    return carry

                lax.fori_loop(0, h // 128, mul_body, 0)
            pltpu.sync_copy(rows_v, out_hbm.at[pl.ds(base, rpw)])

    return gather_k


def kernel(query, memory_keys, memory_values, Wq, Wk):
    B, S, H = query.shape
    M = memory_keys.shape[1]
    KD = Wq.shape[0]
    scale = KD ** (-0.5)
    ch = M // _NC

    qm = pl.pallas_call(
        _proj_body,
        grid=(B,),
        in_specs=[
            pl.BlockSpec((1, S, H), lambda b: (b, 0, 0)),
            pl.BlockSpec((KD, H), lambda b: (0, 0)),
        ],
        out_specs=pl.BlockSpec((1, 1, KD), lambda b: (b, 0, 0)),
        out_shape=jax.ShapeDtypeStruct((B, 1, KD), jnp.float32),
    )(query, Wq)

    scores = pl.pallas_call(
        functools.partial(_scores_body, scale=scale),
        grid=(B, _NC),
        in_specs=[
            pl.BlockSpec((1, 1, KD), lambda b, j: (b, 0, 0)),
            pl.BlockSpec((KD, H), lambda b, j: (0, 0)),
            pl.BlockSpec((1, ch, H), lambda b, j: (b, j, 0)),
        ],
        out_specs=pl.BlockSpec((1, ch // 128, 128), lambda b, j: (b, j, 0)),
        out_shape=jax.ShapeDtypeStruct((B, M // 128, 128), jnp.float32),
    )(qm, Wk, memory_keys)

    idx_all, w_all, wrep = pl.pallas_call(
        functools.partial(_topk_body, m_total=M, n_batch=B),
        out_shape=[
            jax.ShapeDtypeStruct((1, 128), jnp.int32),
            jax.ShapeDtypeStruct((1, 128), jnp.float32),
            jax.ShapeDtypeStruct((B * _TOPK, 16), jnp.float32),
        ],
    )(scores)

    idx_flat = idx_all.reshape(B * _TOPK)
    values_flat = memory_values.reshape(B * M, H)

    out = _make_sc_gather(B * _TOPK, H)(values_flat, idx_flat, wrep)
    return out.reshape(B, _TOPK, H)
